# Initial kernel scaffold; baseline (speedup 1.0000x reference)
import jax
import jax.numpy as jnp
from jax.experimental import pallas as pl


def kernel(uv_inputs, texture_id, data):
    raise NotImplementedError("write your pallas kernel here")


# speedup vs baseline: 60.9691x; 60.9691x over previous
import functools

import jax
import jax.numpy as jnp
from jax import lax
from jax.experimental import pallas as pl
from jax.experimental.pallas import tpu as pltpu
from jax.experimental.pallas import tpu_sc as plsc

H = W = 512
P = H * W
C = 16
TH, TW = 1024, 512
NC, NS, L = 2, 16, 16
NW = NC * NS
PPW = P // NW
CH = 128
NCHUNK = PPW // CH
G = CH // L

LEVELS = ((0, 512), (512, 256), (768, 128), (896, 64))


def _body(ux_hbm, uy_hbm, table_hbm, out_hbm,
          ux_v, uy_v,
          i00_v, i01_v, i10_v, i11_v,
          w00_v, w01_v, w10_v, w11_v,
          t00_v, t01_v, t10_v, t11_v,
          acc_v, sem):
    cid = lax.axis_index("c")
    sid = lax.axis_index("s")
    wid = sid * NC + cid
    pbase = wid * PPW
    pltpu.sync_copy(ux_hbm.at[pl.ds(pbase, PPW)], ux_v)
    pltpu.sync_copy(uy_hbm.at[pl.ds(pbase, PPW)], uy_v)
    lanes = lax.iota(jnp.int32, L)

    def chunk_body(ch, carry):
        cbase = ch * CH
        for lvl in range(4):
            off_y, w = LEVELS[lvl]

            def build(g, c2):
                s = cbase + g * L
                gx = ux_v[pl.ds(s, L)]
                gy = uy_v[pl.ds(s, L)]
                x = ((gx + 1.0) * w - 1.0) * 0.5
                y = ((gy + 1.0) * w - 1.0) * 0.5
                x = jnp.minimum(jnp.maximum(x, 0.0), w - 1.0)
                y = jnp.minimum(jnp.maximum(y, 0.0), w - 1.0)
                xi = x.astype(jnp.int32)
                yi = y.astype(jnp.int32)
                wx1 = x - xi.astype(jnp.float32)
                wy1 = y - yi.astype(jnp.float32)
                wx0 = 1.0 - wx1
                wy0 = 1.0 - wy1
                base_idx = (yi + off_y) * TW + xi
                gs = pl.ds(g * L, L)
                i00_v[gs] = base_idx
                i01_v[gs] = base_idx + 1
                i10_v[gs] = base_idx + TW
                i11_v[gs] = base_idx + (TW + 1)
                w00_v[gs] = wy0 * wx0
                w01_v[gs] = wy0 * wx1
                w10_v[gs] = wy1 * wx0
                w11_v[gs] = wy1 * wx1
                return c2

            lax.fori_loop(0, G, build, 0)

            cp0 = pltpu.async_copy(table_hbm.at[i00_v], t00_v, sem)
            cp1 = pltpu.async_copy(table_hbm.at[i01_v], t01_v, sem)
            cp2 = pltpu.async_copy(table_hbm.at[i10_v], t10_v, sem)
            cp3 = pltpu.async_copy(table_hbm.at[i11_v], t11_v, sem)
            cp0.wait()
            cp1.wait()
            cp2.wait()
            cp3.wait()

            def combine(g, c2):
                gs = pl.ds(g * L, L)
                w00 = w00_v[gs]
                w01 = w01_v[gs]
                w10 = w10_v[gs]
                w11 = w11_v[gs]

                def bcast(wv, i):
                    s = lax.squeeze(lax.slice(wv, (i,), (i + 1,)), (0,))
                    return jnp.broadcast_to(s, (L,))

                for i in range(L):
                    pix = g * L + i
                    a = (bcast(w00, i) * t00_v[pix]
                         + bcast(w01, i) * t01_v[pix]
                         + bcast(w10, i) * t10_v[pix]
                         + bcast(w11, i) * t11_v[pix])
                    if lvl == 0:
                        acc_v[pix] = a
                    else:
                        plsc.addupdate(acc_v.at[pix], a)
                return c2

            lax.fori_loop(0, G, combine, 0)

        pltpu.sync_copy(acc_v, out_hbm.at[pl.ds(pbase + cbase, CH)])
        return carry

    lax.fori_loop(0, NCHUNK, chunk_body, 0)


@functools.partial(jax.jit, static_argnames=())
def kernel(uv_inputs, texture_id, data):
    tex = lax.dynamic_index_in_dim(data, texture_id, axis=0, keepdims=False)
    table = jnp.transpose(tex, (1, 2, 0)).reshape(TH * TW, C)
    ux = uv_inputs[0, 0].reshape(P)
    uy = uv_inputs[0, 1].reshape(P)

    mesh = plsc.VectorSubcoreMesh(core_axis_name="c", subcore_axis_name="s")
    run = pl.kernel(
        _body,
        out_type=jax.ShapeDtypeStruct((P, C), jnp.float32),
        mesh=mesh,
        scratch_types=[
            pltpu.VMEM((PPW,), jnp.float32),
            pltpu.VMEM((PPW,), jnp.float32),
            pltpu.VMEM((CH,), jnp.int32),
            pltpu.VMEM((CH,), jnp.int32),
            pltpu.VMEM((CH,), jnp.int32),
            pltpu.VMEM((CH,), jnp.int32),
            pltpu.VMEM((CH,), jnp.float32),
            pltpu.VMEM((CH,), jnp.float32),
            pltpu.VMEM((CH,), jnp.float32),
            pltpu.VMEM((CH,), jnp.float32),
            pltpu.VMEM((CH, C), jnp.float32),
            pltpu.VMEM((CH, C), jnp.float32),
            pltpu.VMEM((CH, C), jnp.float32),
            pltpu.VMEM((CH, C), jnp.float32),
            pltpu.VMEM((CH, C), jnp.float32),
            pltpu.SemaphoreType.DMA,
        ],
        compiler_params=pltpu.CompilerParams(use_tc_tiling_on_sc=False),
    )
    out = run(ux, uy, table)
    return jnp.transpose(out.reshape(1, H, W, C), (0, 3, 1, 2))


# speedup vs baseline: 83.6367x; 1.3718x over previous
import jax
import jax.numpy as jnp
from jax import lax
from jax.experimental import pallas as pl
from jax.experimental.pallas import tpu as pltpu
from jax.experimental.pallas import tpu_sc as plsc

H = W = 512
P = H * W
C = 16
TH, TW = 1024, 512
NC, NS, L = 2, 16, 16
NW = NC * NS
PPW = P // NW
CH = 128
NCHUNK = PPW // CH
G = CH // L

LEVELS = ((0, 512), (512, 256), (768, 128), (896, 64))


def _body(ux_hbm, uy_hbm, table_hbm, out_hbm,
          ux_v, uy_v,
          i00a, i01a, i10a, i11a, wxa, wya, t00a, t01a, t10a, t11a,
          i00b, i01b, i10b, i11b, wxb, wyb, t00b, t01b, t10b, t11b,
          acc_v, sem_a, sem_b):
    bufs = ((i00a, i01a, i10a, i11a, wxa, wya, t00a, t01a, t10a, t11a, sem_a),
            (i00b, i01b, i10b, i11b, wxb, wyb, t00b, t01b, t10b, t11b, sem_b))

    cid = lax.axis_index("c")
    sid = lax.axis_index("s")
    wid = sid * NC + cid
    pbase = wid * PPW
    pltpu.sync_copy(ux_hbm.at[pl.ds(pbase, PPW)], ux_v)
    pltpu.sync_copy(uy_hbm.at[pl.ds(pbase, PPW)], uy_v)

    def build_and_fire(bset, base, lvl):
        i00, i01, i10, i11, wx, wy, t00, t01, t10, t11, sem = bset
        off_y, w = LEVELS[lvl]

        def build_g(g, c2):
            s = base + g * L
            gx = ux_v[pl.ds(s, L)]
            gy = uy_v[pl.ds(s, L)]
            x = ((gx + 1.0) * w - 1.0) * 0.5
            y = ((gy + 1.0) * w - 1.0) * 0.5
            x = jnp.minimum(jnp.maximum(x, 0.0), w - 1.0)
            y = jnp.minimum(jnp.maximum(y, 0.0), w - 1.0)
            xi = x.astype(jnp.int32)
            yi = y.astype(jnp.int32)
            gs = pl.ds(g * L, L)
            b = (yi + off_y) * TW + xi
            i00[gs] = b
            i01[gs] = b + 1
            i10[gs] = b + TW
            i11[gs] = b + (TW + 1)
            wx[gs] = x - xi.astype(jnp.float32)
            wy[gs] = y - yi.astype(jnp.float32)
            return c2

        lax.fori_loop(0, G, build_g, 0)
        pltpu.async_copy(table_hbm.at[i00], t00, sem)
        pltpu.async_copy(table_hbm.at[i01], t01, sem)
        pltpu.async_copy(table_hbm.at[i10], t10, sem)
        pltpu.async_copy(table_hbm.at[i11], t11, sem)

    def wait_and_combine(bset, lvl):
        i00, i01, i10, i11, wx, wy, t00, t01, t10, t11, sem = bset
        pltpu.make_async_copy(table_hbm.at[i00], t00, sem).wait()
        pltpu.make_async_copy(table_hbm.at[i01], t01, sem).wait()
        pltpu.make_async_copy(table_hbm.at[i10], t10, sem).wait()
        pltpu.make_async_copy(table_hbm.at[i11], t11, sem).wait()

        def comb_g(g, c2):
            gs = pl.ds(g * L, L)
            wxv = wx[gs]
            wyv = wy[gs]

            def bcast(wv, i):
                s = lax.squeeze(lax.slice(wv, (i,), (i + 1,)), (0,))
                return jnp.broadcast_to(s, (L,))

            for i in range(L):
                pix = g * L + i
                fx = bcast(wxv, i)
                fy = bcast(wyv, i)
                a00 = t00[pix]
                a01 = t01[pix]
                a10 = t10[pix]
                a11 = t11[pix]
                top = a00 + fx * (a01 - a00)
                bot = a10 + fx * (a11 - a10)
                a = top + fy * (bot - top)
                if lvl == 0:
                    acc_v[pix] = a
                else:
                    plsc.addupdate(acc_v.at[pix], a)
            return c2

        lax.fori_loop(0, G, comb_g, 0)

    build_and_fire(bufs[0], 0, 0)

    def chunk_body(ch, carry):
        cbase = ch * CH
        for lvl in range(4):
            p, q = lvl % 2, (lvl + 1) % 2
            if lvl < 3:
                build_and_fire(bufs[q], cbase, lvl + 1)
            else:
                @pl.when(ch < NCHUNK - 1)
                def _prefetch():
                    build_and_fire(bufs[q], cbase + CH, 0)
            wait_and_combine(bufs[p], lvl)
        pltpu.sync_copy(acc_v, out_hbm.at[pl.ds(pbase + cbase, CH)])
        return carry

    lax.fori_loop(0, NCHUNK, chunk_body, 0)


def kernel(uv_inputs, texture_id, data):
    tex = lax.dynamic_index_in_dim(data, texture_id, axis=0, keepdims=False)
    table = jnp.transpose(tex, (1, 2, 0)).reshape(TH * TW, C)
    ux = uv_inputs[0, 0].reshape(P)
    uy = uv_inputs[0, 1].reshape(P)

    mesh = plsc.VectorSubcoreMesh(core_axis_name="c", subcore_axis_name="s")
    dbl = [
        pltpu.VMEM((CH,), jnp.int32),
        pltpu.VMEM((CH,), jnp.int32),
        pltpu.VMEM((CH,), jnp.int32),
        pltpu.VMEM((CH,), jnp.int32),
        pltpu.VMEM((CH,), jnp.float32),
        pltpu.VMEM((CH,), jnp.float32),
        pltpu.VMEM((CH, C), jnp.float32),
        pltpu.VMEM((CH, C), jnp.float32),
        pltpu.VMEM((CH, C), jnp.float32),
        pltpu.VMEM((CH, C), jnp.float32),
    ]
    run = pl.kernel(
        _body,
        out_type=jax.ShapeDtypeStruct((P, C), jnp.float32),
        mesh=mesh,
        scratch_types=(
            [pltpu.VMEM((PPW,), jnp.float32), pltpu.VMEM((PPW,), jnp.float32)]
            + dbl + dbl
            + [pltpu.VMEM((CH, C), jnp.float32),
               pltpu.SemaphoreType.DMA, pltpu.SemaphoreType.DMA]
        ),
        compiler_params=pltpu.CompilerParams(use_tc_tiling_on_sc=False),
    )
    out = run(ux, uy, table)
    return jnp.transpose(out.reshape(1, H, W, C), (0, 3, 1, 2))
